# baseline (speedup 1.0000x reference)
import functools

import jax
import jax.numpy as jnp
from jax.experimental import pallas as pl

D = 64
H = 6
NA = 64
NL = 192
T = 20
L = 10
NM = 6
PLEN = 30
N = NA + NL
WPACK = 384

_F32 = jnp.float32


def _mm(x, w):
    return jax.lax.dot_general(
        x, w, (((1,), (0,)), ((), ())), preferred_element_type=_F32
    )


def _mm_t(x, w):
    return jax.lax.dot_general(
        x, w, (((1,), (1,)), ((), ())), preferred_element_type=_F32
    )


def _ln(x):
    m = x.mean(-1, keepdims=True)
    v = ((x - m) ** 2).mean(-1, keepdims=True)
    return (x - m) * jax.lax.rsqrt(v + 1e-5)


def _lres(x, w1, w2, ws=None):
    y = jax.nn.relu(_ln(_mm(x, w1)))
    y = _ln(_mm(y, w2))
    sk = _mm(x, ws) if ws is not None else x
    return jax.nn.relu(y + sk)


def _vn_kernel(adata, ldata, wref, preds_ref, cls_ref, *, metas):
    w_iter = iter(metas)

    def nxt():
        off, r, c = next(w_iter)
        return wref[off:off + r, :c]

    (l1w1, l1w2, l1ws, l2w1, l2w2, l2ws, wih, whh, bih, bhh, h0, c0,
     sgw1, sgw2, sgws, b1w1, b1w2, b1ws, b2w1, b2w2, fw1, fw2,
     wq, wk, wv, wo1, wo2, wl1, wl2) = (nxt() for _ in range(29))
    mode_ws = [nxt() for _ in range(4 * NM)]
    goal_w, srw1, srw2, srws, s_out = (nxt() for _ in range(5))

    ad = adata[...]
    traj_v = ad[:, :2 * T]
    pad_v = ad[:, 2 * T:]
    blocks = []
    for t in range(T):
        xy_t = traj_v[:, 2 * t:2 * t + 2]
        pad_t = pad_v[:, t:t + 1]
        if t == 0:
            feats_t = jnp.zeros_like(xy_t)
            m_t = pad_t
        else:
            feats_t = xy_t - traj_v[:, 2 * t - 2:2 * t]
            m_t = pad_t * pad_v[:, t - 1:t]
        core_t = jnp.concatenate([xy_t, feats_t], axis=-1) * m_t
        blocks.append(jnp.concatenate([core_t, pad_t], axis=-1))
    x_in = jnp.concatenate(blocks, axis=0)

    x = _lres(x_in, l1w1, l1w2, l1ws)
    x = _lres(x, l2w1, l2w2, l2ws)
    h = jnp.broadcast_to(h0, (NA, D))
    c = jnp.broadcast_to(c0, (NA, D))
    b_v = bih + bhh
    for t in range(T):
        xt = x[t * NA:(t + 1) * NA]
        g = _mm(xt, wih) + b_v + _mm(h, whh)
        gi = jax.nn.sigmoid(g[:, :D])
        gf = jax.nn.sigmoid(g[:, D:2 * D])
        gg = jnp.tanh(g[:, 2 * D:3 * D])
        go = jax.nn.sigmoid(g[:, 3 * D:])
        c = gf * c + gi * gg
        h = go * jnp.tanh(c)
    agents = h

    ld = ldata[...]
    l1 = [_lres(ld[:, 2 * v:2 * v + 2], b1w1, b1w2, b1ws) for v in range(L)]
    m1 = l1[0]
    for v in range(1, L):
        m1 = jnp.maximum(m1, l1[v])
    m2 = None
    for v in range(L):
        l2v = _lres(jnp.concatenate([l1[v], m1], axis=-1), b2w1, b2w2)
        m2 = l2v if m2 is None else jnp.maximum(m2, l2v)
    lfeat = jnp.concatenate([m2, m2], axis=-1)
    seg = _lres(ld[:, 2 * L:2 * L + 4], sgw1, sgw2, sgws)
    lanes = _lres(lfeat + seg, fw1, fw2)

    nodes = jnp.concatenate([agents, lanes], axis=0)
    q = _ln(_mm(nodes, wq))
    k = _ln(_mm(nodes, wk))
    vv = jax.nn.relu(_ln(_mm(nodes, wv)))
    outs = []
    scale = D ** -0.5
    for hh in range(H):
        qh = q[:, hh * D:(hh + 1) * D]
        kh = k[:, hh * D:(hh + 1) * D]
        vh = vv[:, hh * D:(hh + 1) * D]
        e = jnp.exp(_mm_t(qh, kh) * scale)
        att = e / jnp.sum(e, axis=-1, keepdims=True)
        outs.append(_mm(att, vh))
    out = jnp.concatenate(outs, axis=-1)
    out = _mm(jax.nn.relu(_ln(_mm(out, wo1))), wo2)
    nd = jax.nn.relu(_ln(_mm(nodes, wl1) + out))
    nd = jax.nn.relu(_mm(nd, wl2) + nodes)
    agents_i = nd[:NA]

    a2 = jnp.concatenate([agents, agents_i], axis=-1)
    preds = []
    for m in range(NM):
        mw1, mw2, mws, mout = mode_ws[4 * m:4 * m + 4]
        pm = _mm(_lres(a2, mw1, mw2, mws), mout)
        preds.append(pm)
    preds_cat = jnp.concatenate(preds, axis=0)
    preds_ref[...] = preds_cat

    r_idx = jax.lax.broadcasted_iota(jnp.int32, (NA * NM, NA), 0)
    a_idx = jax.lax.broadcasted_iota(jnp.int32, (NA * NM, NA), 1)
    dd = r_idx - a_idx * NM
    rep_v = ((dd >= 0) & (dd < NM)).astype(_F32)
    goals = preds_cat[:, 58:60]
    gemb = jax.nn.relu(_ln(_mm(goals, goal_w)))
    a2r = _mm(rep_v, a2)
    ag = jnp.concatenate([a2r, gemb], axis=-1)
    s = _mm(_lres(ag, srw1, srw2, srws), s_out)
    e = jnp.exp(jnp.maximum(s - jnp.max(s), -60.0))
    gsum = jax.lax.dot_general(
        rep_v, e, (((0,), (0,)), ((), ())), preferred_element_type=_F32
    )
    cls_ref[...] = e / _mm(rep_v, gsum)


def _pack(ws):
    blocks, metas = [], []
    off = 0
    for w in ws:
        if w.ndim == 1:
            w = w.reshape(1, -1)
        r, c = w.shape
        rp = (-r) % 8
        blocks.append(jnp.pad(w, ((0, rp), (0, WPACK - c))))
        metas.append((off, r, c))
        off += r + rp
    return jnp.concatenate(blocks, axis=0), tuple(metas)


@jax.jit
def kernel(trajs_obs, pad_obs, lane_feats, turn, control, intersect, params):
    pa, plane, pi, pd = params['agent'], params['lane'], params['inter'], params['dec']
    ws = [
        pa['lr1']['W1'], pa['lr1']['W2'], pa['lr1']['Ws'],
        pa['lr2']['W1'], pa['lr2']['W2'], pa['lr2']['Ws'],
        pa['Wih'], pa['Whh'], pa['bih'], pa['bhh'], pa['h0'], pa['c0'],
        plane['seg']['W1'], plane['seg']['W2'], plane['seg']['Ws'],
        plane['bb1']['W1'], plane['bb1']['W2'], plane['bb1']['Ws'],
        plane['bb2']['W1'], plane['bb2']['W2'],
        plane['fuse']['W1'], plane['fuse']['W2'],
        pi['Wq'], pi['Wk'], pi['Wv'], pi['Wo1'], pi['Wo2'],
        pi['Wl1'], pi['Wl2'],
    ]
    for m in pd['modes']:
        ws += [m['res']['W1'], m['res']['W2'], m['res']['Ws'], m['out']]
    ws += [pd['goal'], pd['score_res']['W1'], pd['score_res']['W2'],
           pd['score_res']['Ws'], pd['score_out']]
    packed, metas = _pack(ws)

    adata = jnp.concatenate([trajs_obs.reshape(NA, 2 * T), pad_obs], axis=-1)
    ldata = jnp.concatenate(
        [lane_feats.reshape(NL, 2 * L), turn, control[:, None],
         intersect[:, None]], axis=-1)

    preds_cat, cls_flat = pl.pallas_call(
        functools.partial(_vn_kernel, metas=metas),
        out_shape=[
            jax.ShapeDtypeStruct((NM * NA, 2 * PLEN), _F32),
            jax.ShapeDtypeStruct((NA * NM, 1), _F32),
        ],
    )(adata, ldata, packed)

    reg = preds_cat.reshape(NA, NM, PLEN, 2)
    cls = cls_flat.reshape(NA, NM)
    return (reg, cls)


# speedup vs baseline: 1.5741x; 1.5741x over previous
import jax
import jax.numpy as jnp
import numpy as np
from jax.experimental import pallas as pl

D = 64
H = 6
NA = 64
NL = 192
T = 20
L = 10
NM = 6
PLEN = 30
N = NA + NL

_F32 = jnp.float32


def _mm(x, w):
    return jax.lax.dot_general(
        x, w, (((1,), (0,)), ((), ())), preferred_element_type=_F32
    )


def _mm_t(x, w):
    return jax.lax.dot_general(
        x, w, (((1,), (1,)), ((), ())), preferred_element_type=_F32
    )


def _ln(x):
    c = x.shape[-1]
    ones = jnp.ones((c, 1), _F32)
    m = _mm(x, ones) * (1.0 / c)
    ex2 = _mm(x * x, ones) * (1.0 / c)
    v = ex2 - m * m
    return (x - m) * jax.lax.rsqrt(v + 1e-5)


def _lres(x, w1, w2, ws=None):
    y = jax.nn.relu(_ln(_mm(x, w1)))
    y = _ln(_mm(y, w2))
    sk = _mm(x, ws) if ws is not None else x
    return jax.nn.relu(y + sk)


def _vn_kernel(*refs):
    (x_tm, lane_v, lane_seg, rep,
     l1w1, l1w2, l1ws, l2w1, l2w2, l2ws,
     wih, whh, bih, bhh, h0, c0,
     sgw1, sgw2, sgws, b1w1, b1w2, b1ws, b2w1, b2w2, fw1, fw2,
     wq, wk, wv, wo1, wo2, wl1, wl2) = refs[:33]
    mode_ws = refs[33:33 + 4 * NM]
    goal_w, srw1, srw2, srws, s_out = refs[33 + 4 * NM:33 + 4 * NM + 5]
    preds_ref, cls_ref = refs[-2:]

    x = _lres(x_tm[...], l1w1[...], l1w2[...], l1ws[...])
    x = _lres(x, l2w1[...], l2w2[...], l2ws[...])
    xw = _mm(x, wih[...]) + (bih[...] + bhh[...])
    h = jnp.broadcast_to(h0[...], (NA, D))
    c = jnp.broadcast_to(c0[...], (NA, D))
    whh_v = whh[...]
    for t in range(T):
        g = xw[t * NA:(t + 1) * NA] + _mm(h, whh_v)
        gi = jax.nn.sigmoid(g[:, :D])
        gf = jax.nn.sigmoid(g[:, D:2 * D])
        gg = jnp.tanh(g[:, 2 * D:3 * D])
        go = jax.nn.sigmoid(g[:, 3 * D:])
        c = gf * c + gi * gg
        h = go * jnp.tanh(c)
    agents = h

    l1 = [_lres(lane_v[v], b1w1[...], b1w2[...], b1ws[...]) for v in range(L)]
    m1 = l1[0]
    for v in range(1, L):
        m1 = jnp.maximum(m1, l1[v])
    m2 = None
    for v in range(L):
        l2v = _lres(jnp.concatenate([l1[v], m1], axis=-1), b2w1[...], b2w2[...])
        m2 = l2v if m2 is None else jnp.maximum(m2, l2v)
    lfeat = jnp.concatenate([m2, m2], axis=-1)
    seg = _lres(lane_seg[...], sgw1[...], sgw2[...], sgws[...])
    lanes = _lres(lfeat + seg, fw1[...], fw2[...])

    nodes = jnp.concatenate([agents, lanes], axis=0)
    q = _ln(_mm(nodes, wq[...]))
    k = _ln(_mm(nodes, wk[...]))
    vv = jax.nn.relu(_ln(_mm(nodes, wv[...])))
    ones_col = jnp.ones((N, 1), _F32)
    outs = []
    scale = D ** -0.5
    for hh in range(H):
        qh = q[:, hh * D:(hh + 1) * D]
        kh = k[:, hh * D:(hh + 1) * D]
        vh = jnp.concatenate([vv[:, hh * D:(hh + 1) * D], ones_col], axis=-1)
        e = jnp.exp(_mm_t(qh, kh) * scale)
        av = _mm(e, vh)
        outs.append(av[:, :D] / av[:, D:D + 1])
    out = jnp.concatenate(outs, axis=-1)
    out = _mm(jax.nn.relu(_ln(_mm(out, wo1[...]))), wo2[...])
    nd = jax.nn.relu(_ln(_mm(nodes, wl1[...]) + out))
    nd = jax.nn.relu(_mm(nd, wl2[...]) + nodes)
    agents_i = nd[:NA]

    a2 = jnp.concatenate([agents, agents_i], axis=-1)
    preds = []
    for m in range(NM):
        mw1, mw2, mws, mout = mode_ws[4 * m:4 * m + 4]
        pm = _mm(_lres(a2, mw1[...], mw2[...], mws[...]), mout[...])
        preds.append(pm)
    preds_cat = jnp.concatenate(preds, axis=0)
    preds_ref[...] = preds_cat

    rep_v = rep[...]
    goals = preds_cat[:, 58:60]
    gemb = jax.nn.relu(_ln(_mm(goals, goal_w[...])))
    a2r = _mm(rep_v, a2)
    ag = jnp.concatenate([a2r, gemb], axis=-1)
    s = _mm(_lres(ag, srw1[...], srw2[...], srws[...]), s_out[...])
    e = jnp.exp(jnp.maximum(s - jnp.max(s), -60.0))
    gsum = jax.lax.dot_general(
        rep_v, e, (((0,), (0,)), ((), ())), preferred_element_type=_F32
    )
    cls_ref[...] = e / _mm(rep_v, gsum)


@jax.jit
def kernel(trajs_obs, pad_obs, lane_feats, turn, control, intersect, params):
    xy = trajs_obs[:, :, :2]
    feats = jnp.concatenate([jnp.zeros_like(xy[:, :1]), xy[:, 1:] - xy[:, :-1]],
                            axis=1)
    a = jnp.concatenate([xy, feats, pad_obs[..., None]], axis=-1)
    core = a[:, :, :-1] * a[:, :, -1:]
    core = jnp.concatenate([core[:, :1], core[:, 1:] * a[:, :-1, -1:]], axis=1)
    agents_in = jnp.concatenate([core, a[:, :, -1:]], axis=-1)
    x_tm = agents_in.transpose(1, 0, 2).reshape(T * NA, 5)
    lane_v = lane_feats.transpose(1, 0, 2)
    lane_seg = jnp.concatenate(
        [turn, control[:, None], intersect[:, None]], axis=-1)

    rep_np = np.zeros((NA * NM, NA), np.float32)
    rep_np[np.arange(NA * NM), np.arange(NA * NM) // NM] = 1.0
    rep = jnp.asarray(rep_np)

    pa, plane, pi, pd = params['agent'], params['lane'], params['inter'], params['dec']
    row = lambda b: b.reshape(1, -1)
    ins = [
        x_tm, lane_v, lane_seg, rep,
        pa['lr1']['W1'], pa['lr1']['W2'], pa['lr1']['Ws'],
        pa['lr2']['W1'], pa['lr2']['W2'], pa['lr2']['Ws'],
        pa['Wih'], pa['Whh'], row(pa['bih']), row(pa['bhh']),
        row(pa['h0']), row(pa['c0']),
        plane['seg']['W1'], plane['seg']['W2'], plane['seg']['Ws'],
        plane['bb1']['W1'], plane['bb1']['W2'], plane['bb1']['Ws'],
        plane['bb2']['W1'], plane['bb2']['W2'],
        plane['fuse']['W1'], plane['fuse']['W2'],
        pi['Wq'], pi['Wk'], pi['Wv'], pi['Wo1'], pi['Wo2'],
        pi['Wl1'], pi['Wl2'],
    ]
    for m in pd['modes']:
        ins += [m['res']['W1'], m['res']['W2'], m['res']['Ws'], m['out']]
    ins += [pd['goal'], pd['score_res']['W1'], pd['score_res']['W2'],
            pd['score_res']['Ws'], pd['score_out']]

    preds_cat, cls_flat = pl.pallas_call(
        _vn_kernel,
        out_shape=[
            jax.ShapeDtypeStruct((NM * NA, 2 * PLEN), _F32),
            jax.ShapeDtypeStruct((NA * NM, 1), _F32),
        ],
    )(*ins)

    reg = preds_cat.reshape(NA, NM, PLEN, 2)
    cls = cls_flat.reshape(NA, NM)
    return (reg, cls)


# speedup vs baseline: 1.7773x; 1.1291x over previous
import jax
import jax.numpy as jnp
import numpy as np
from jax.experimental import pallas as pl

D = 64
H = 6
NA = 64
NL = 192
T = 20
L = 10
NM = 6
PLEN = 30
N = NA + NL

_F32 = jnp.float32


def _mm(x, w):
    return jax.lax.dot_general(
        x, w, (((1,), (0,)), ((), ())), preferred_element_type=_F32
    )


def _mm_t(x, w):
    return jax.lax.dot_general(
        x, w, (((1,), (1,)), ((), ())), preferred_element_type=_F32
    )


def _ln(x):
    m = x.mean(-1, keepdims=True)
    v = ((x - m) ** 2).mean(-1, keepdims=True)
    return (x - m) * jax.lax.rsqrt(v + 1e-5)


def _lres(x, w1, w2, ws=None):
    y = jax.nn.relu(_ln(_mm(x, w1)))
    y = _ln(_mm(y, w2))
    sk = _mm(x, ws) if ws is not None else x
    return jax.nn.relu(y + sk)


def _vn_kernel(*refs):
    (x_tm, lane_v, lane_seg, rep,
     l1w1, l1w2, l1ws, l2w1, l2w2, l2ws,
     wih, whh, bih, bhh, h0, c0,
     sgw1, sgw2, sgws, b1w1, b1w2, b1ws, b2w1, b2w2, fw1, fw2,
     wq, wk, wv, wo1, wo2, wl1, wl2) = refs[:33]
    mode_ws = refs[33:33 + 4 * NM]
    goal_w, srw1, srw2, srws, s_out = refs[33 + 4 * NM:33 + 4 * NM + 5]
    preds_ref, cls_ref = refs[-2:]

    x = _lres(x_tm[...], l1w1[...], l1w2[...], l1ws[...])
    x = _lres(x, l2w1[...], l2w2[...], l2ws[...])
    xw = _mm(x, wih[...]) + (bih[...] + bhh[...])
    h = jnp.broadcast_to(h0[...], (NA, D))
    c = jnp.broadcast_to(c0[...], (NA, D))
    whh_v = whh[...]
    for t in range(T):
        g = xw[t * NA:(t + 1) * NA] + _mm(h, whh_v)
        gi = jax.nn.sigmoid(g[:, :D])
        gf = jax.nn.sigmoid(g[:, D:2 * D])
        gg = jnp.tanh(g[:, 2 * D:3 * D])
        go = jax.nn.sigmoid(g[:, 3 * D:])
        c = gf * c + gi * gg
        h = go * jnp.tanh(c)
    agents = h

    l1 = [_lres(lane_v[v], b1w1[...], b1w2[...], b1ws[...]) for v in range(L)]
    m1 = l1[0]
    for v in range(1, L):
        m1 = jnp.maximum(m1, l1[v])
    m2 = None
    for v in range(L):
        l2v = _lres(jnp.concatenate([l1[v], m1], axis=-1), b2w1[...], b2w2[...])
        m2 = l2v if m2 is None else jnp.maximum(m2, l2v)
    lfeat = jnp.concatenate([m2, m2], axis=-1)
    seg = _lres(lane_seg[...], sgw1[...], sgw2[...], sgws[...])
    lanes = _lres(lfeat + seg, fw1[...], fw2[...])

    nodes = jnp.concatenate([agents, lanes], axis=0)
    q = _ln(_mm(nodes, wq[...]))
    k = _ln(_mm(nodes, wk[...]))
    vv = jax.nn.relu(_ln(_mm(nodes, wv[...])))
    ones_col = jnp.ones((N, 1), _F32)
    outs = []
    scale = D ** -0.5
    for hh in range(H):
        qh = q[:, hh * D:(hh + 1) * D]
        kh = k[:, hh * D:(hh + 1) * D]
        vh = jnp.concatenate([vv[:, hh * D:(hh + 1) * D], ones_col], axis=-1)
        e = jnp.exp(_mm_t(qh, kh) * scale)
        av = _mm(e, vh)
        outs.append(av[:, :D] / av[:, D:D + 1])
    out = jnp.concatenate(outs, axis=-1)
    out = _mm(jax.nn.relu(_ln(_mm(out, wo1[...]))), wo2[...])
    nd = jax.nn.relu(_ln(_mm(nodes, wl1[...]) + out))
    nd = jax.nn.relu(_mm(nd, wl2[...]) + nodes)
    agents_i = nd[:NA]

    a2 = jnp.concatenate([agents, agents_i], axis=-1)
    preds = []
    for m in range(NM):
        mw1, mw2, mws, mout = mode_ws[4 * m:4 * m + 4]
        pm = _mm(_lres(a2, mw1[...], mw2[...], mws[...]), mout[...])
        preds.append(pm)
    preds_cat = jnp.concatenate(preds, axis=0)
    preds_ref[...] = preds_cat

    rep_v = rep[...]
    goals = preds_cat[:, 58:60]
    gemb = jax.nn.relu(_ln(_mm(goals, goal_w[...])))
    a2r = _mm(rep_v, a2)
    ag = jnp.concatenate([a2r, gemb], axis=-1)
    s = _mm(_lres(ag, srw1[...], srw2[...], srws[...]), s_out[...])
    e = jnp.exp(jnp.maximum(s - jnp.max(s), -60.0))
    gsum = jax.lax.dot_general(
        rep_v, e, (((0,), (0,)), ((), ())), preferred_element_type=_F32
    )
    cls_ref[...] = e / _mm(rep_v, gsum)


@jax.jit
def kernel(trajs_obs, pad_obs, lane_feats, turn, control, intersect, params):
    xy = trajs_obs[:, :, :2]
    feats = jnp.concatenate([jnp.zeros_like(xy[:, :1]), xy[:, 1:] - xy[:, :-1]],
                            axis=1)
    a = jnp.concatenate([xy, feats, pad_obs[..., None]], axis=-1)
    core = a[:, :, :-1] * a[:, :, -1:]
    core = jnp.concatenate([core[:, :1], core[:, 1:] * a[:, :-1, -1:]], axis=1)
    agents_in = jnp.concatenate([core, a[:, :, -1:]], axis=-1)
    x_tm = agents_in.transpose(1, 0, 2).reshape(T * NA, 5)
    lane_v = lane_feats.transpose(1, 0, 2)
    lane_seg = jnp.concatenate(
        [turn, control[:, None], intersect[:, None]], axis=-1)

    rep_np = np.zeros((NA * NM, NA), np.float32)
    rep_np[np.arange(NA * NM), np.arange(NA * NM) // NM] = 1.0
    rep = jnp.asarray(rep_np)

    pa, plane, pi, pd = params['agent'], params['lane'], params['inter'], params['dec']
    row = lambda b: b.reshape(1, -1)
    ins = [
        x_tm, lane_v, lane_seg, rep,
        pa['lr1']['W1'], pa['lr1']['W2'], pa['lr1']['Ws'],
        pa['lr2']['W1'], pa['lr2']['W2'], pa['lr2']['Ws'],
        pa['Wih'], pa['Whh'], row(pa['bih']), row(pa['bhh']),
        row(pa['h0']), row(pa['c0']),
        plane['seg']['W1'], plane['seg']['W2'], plane['seg']['Ws'],
        plane['bb1']['W1'], plane['bb1']['W2'], plane['bb1']['Ws'],
        plane['bb2']['W1'], plane['bb2']['W2'],
        plane['fuse']['W1'], plane['fuse']['W2'],
        pi['Wq'], pi['Wk'], pi['Wv'], pi['Wo1'], pi['Wo2'],
        pi['Wl1'], pi['Wl2'],
    ]
    for m in pd['modes']:
        ins += [m['res']['W1'], m['res']['W2'], m['res']['Ws'], m['out']]
    ins += [pd['goal'], pd['score_res']['W1'], pd['score_res']['W2'],
            pd['score_res']['Ws'], pd['score_out']]

    preds_cat, cls_flat = pl.pallas_call(
        _vn_kernel,
        out_shape=[
            jax.ShapeDtypeStruct((NM * NA, 2 * PLEN), _F32),
            jax.ShapeDtypeStruct((NA * NM, 1), _F32),
        ],
    )(*ins)

    reg = preds_cat.reshape(NA, NM, PLEN, 2)
    cls = cls_flat.reshape(NA, NM)
    return (reg, cls)
